# Initial kernel scaffold; baseline (speedup 1.0000x reference)
import jax
import jax.numpy as jnp
from jax.experimental import pallas as pl


def kernel(xyz, points, params):
    raise NotImplementedError("write your pallas kernel here")


# speedup vs baseline: 1.1574x; 1.1574x over previous
import functools

import jax
import jax.numpy as jnp
from jax import lax
from jax.experimental import pallas as pl
from jax.experimental.pallas import tpu as pltpu

_NPOINT = 1024
_NSAMPLES = (16, 32, 128)
_MLPS = ((32, 32, 64), (64, 64, 128), (64, 96, 128))
_IN_CH = 16
_EPS = 1e-5
_KMAX = 128
_CIN_PAD = 32
_TQ = 16


def _mlp_body(x_ref, *refs):
    ws = refs[0:9]
    bs = refs[9:18]
    o0, o1, o2 = refs[18], refs[19], refs[20]
    x = x_ref[...]

    def chain(xm, s):
        h = xm
        for l in range(3):
            w = ws[3 * s + l][...]
            b = bs[3 * s + l][...]
            h = jnp.dot(h, w, preferred_element_type=jnp.float32) + b
            if l < 2:
                h = jnp.maximum(h, 0.0)
        return h

    tq = x.shape[0]
    x2 = x.reshape(tq * 128, _CIN_PAD)
    y2 = chain(x2, 2).reshape(tq, 128, _MLPS[2][-1])
    o2[...] = jnp.max(y2, axis=1)
    x1 = x[:, :32, :].reshape(tq * 32, _CIN_PAD)
    y1 = chain(x1, 1).reshape(tq, 32, _MLPS[1][-1])
    o1[...] = jnp.max(y1, axis=1)
    x0 = x[:, :16, :].reshape(tq * 16, _CIN_PAD)
    y0 = chain(x0, 0).reshape(tq, 16, _MLPS[0][-1])
    o0[...] = jnp.max(y0, axis=1)


def _fused_mlp(feat, wts, bs):
    m = feat.shape[0]
    grid = (m // _TQ,)
    w_specs = [pl.BlockSpec(w.shape, lambda i: (0, 0)) for w in wts]
    b_specs = [pl.BlockSpec(b.shape, lambda i: (0, 0)) for b in bs]
    outs = [
        jax.ShapeDtypeStruct((m, _MLPS[0][-1]), jnp.float32),
        jax.ShapeDtypeStruct((m, _MLPS[1][-1]), jnp.float32),
        jax.ShapeDtypeStruct((m, _MLPS[2][-1]), jnp.float32),
    ]
    out_specs = [
        pl.BlockSpec((_TQ, _MLPS[0][-1]), lambda i: (i, 0)),
        pl.BlockSpec((_TQ, _MLPS[1][-1]), lambda i: (i, 0)),
        pl.BlockSpec((_TQ, _MLPS[2][-1]), lambda i: (i, 0)),
    ]
    return pl.pallas_call(
        _mlp_body,
        grid=grid,
        in_specs=[pl.BlockSpec((_TQ, _KMAX, _CIN_PAD), lambda i: (i, 0, 0))]
        + w_specs + b_specs,
        out_specs=out_specs,
        out_shape=outs,
    )(feat, *wts, *bs)


def kernel(xyz, points, params):
    B, _, N = xyz.shape
    perm = jax.random.permutation(jax.random.key(42), N)[:_NPOINT]
    new_xyz = xyz[:, :, perm]
    a = jnp.transpose(new_xyz, (0, 2, 1))
    bpts = jnp.transpose(xyz, (0, 2, 1))
    p = jnp.transpose(points, (0, 2, 1))

    d2 = (jnp.sum(a * a, -1)[:, :, None] + jnp.sum(bpts * bpts, -1)[:, None, :]
          - 2.0 * jnp.einsum('bqc,bnc->bqn', a, bpts))
    dist = jnp.sqrt(jnp.maximum(d2, 0.0))
    _, idx = lax.top_k(-dist, _KMAX)

    bidx = jnp.arange(B)[:, None, None]
    gx = bpts[bidx, idx] - a[:, :, None, :]
    gp = p[bidx, idx]
    feat = jnp.concatenate(
        [gx, gp, jnp.zeros((B, _NPOINT, _KMAX, _CIN_PAD - 3 - _IN_CH), jnp.float32)],
        axis=-1,
    ).reshape(B * _NPOINT, _KMAX, _CIN_PAD)

    wts, bs = [], []
    inv = 1.0 / jnp.sqrt(1.0 + _EPS)
    for s in range(3):
        for l in range(3):
            g = params[f"g{s}_{l}"] * inv
            w = params[f"w{s}_{l}"] * g[:, None]
            b = params[f"b{s}_{l}"] * g + params[f"be{s}_{l}"]
            wt = jnp.transpose(w)
            if l == 0:
                wt = jnp.pad(wt, ((0, _CIN_PAD - wt.shape[0]), (0, 0)))
            wts.append(wt)
            bs.append(b[None, :])

    y0, y1, y2 = _fused_mlp(feat, wts, bs)
    out = jnp.concatenate([y0, y1, y2], axis=-1)
    out = jnp.transpose(out.reshape(B, _NPOINT, 320), (0, 2, 1))
    return new_xyz, out
